# Initial kernel scaffold; baseline (speedup 1.0000x reference)
import functools

import jax
import jax.numpy as jnp
from jax import lax
from jax.experimental import pallas as pl
from jax.experimental.pallas import tpu as pltpu
from jax.experimental.pallas import tpu_sc as plsc

WALK_LEN = 50
BATCH = 4096
D = 64
C = 128
NC, NS = 2, 16
NW = NC * NS


def _sc_embed(node_idx, edge_idx, node_table, edge_table, node_pos, edge_pos):
    mesh = plsc.VectorSubcoreMesh(core_axis_name="c", subcore_axis_name="s")

    @functools.partial(
        pl.kernel,
        mesh=mesh,
        out_type=jax.ShapeDtypeStruct((2 * WALK_LEN * BATCH, D), jnp.float32),
        scratch_types=[
            pltpu.VMEM((C,), jnp.int32),
            pltpu.VMEM((C, D), jnp.float32),
            pltpu.VMEM((WALK_LEN, D), jnp.float32),
            pltpu.VMEM((WALK_LEN, D), jnp.float32),
            pltpu.SemaphoreType.DMA,
        ],
    )
    def k(nidx_hbm, eidx_hbm, ntab_hbm, etab_hbm, npos_hbm, epos_hbm,
          out_hbm, idx_v, rows_v, npos_v, epos_v, sem):
        wid = lax.axis_index("s") * NC + lax.axis_index("c")
        pltpu.sync_copy(npos_hbm, npos_v)
        pltpu.sync_copy(epos_hbm, epos_v)

        def do_table(idx_hbm, tab_hbm, pos_v, out_row_off):
            def body(pos, _):
                base = pos * BATCH + wid * C
                pltpu.sync_copy(idx_hbm.at[pl.ds(base, C)], idx_v)
                pltpu.async_copy(tab_hbm.at[idx_v], rows_v, sem).wait()
                pos_vecs = [pos_v[pos, pl.ds(16 * d4, 16)]
                            for d4 in range(D // 16)]

                def row_body(r, _):
                    for d4 in range(D // 16):
                        sl = pl.ds(16 * d4, 16)
                        rows_v[r, sl] = rows_v[r, sl] + pos_vecs[d4]
                    return 0

                lax.fori_loop(0, C, row_body, 0)
                pltpu.sync_copy(rows_v, out_hbm.at[pl.ds(out_row_off + base, C)])
                return 0

            lax.fori_loop(0, WALK_LEN, body, 0)

        do_table(nidx_hbm, ntab_hbm, npos_v, 0)
        do_table(eidx_hbm, etab_hbm, epos_v, WALK_LEN * BATCH)

    return k(node_idx, edge_idx, node_table, edge_table, node_pos, edge_pos)


def kernel(node_idx, edge_idx, node_table, edge_table, node_pos, edge_pos):
    nflat = node_idx.reshape(-1).astype(jnp.int32)
    eflat = edge_idx.reshape(-1).astype(jnp.int32)
    out = _sc_embed(nflat, eflat, node_table, edge_table, node_pos, edge_pos)
    return out.reshape(2 * WALK_LEN, BATCH, D)


# speedup vs baseline: 1.0439x; 1.0439x over previous
import functools

import jax
import jax.numpy as jnp
from jax import lax
from jax.experimental import pallas as pl
from jax.experimental.pallas import tpu as pltpu
from jax.experimental.pallas import tpu_sc as plsc

WALK_LEN = 50
BATCH = 4096
D = 64
C = 128
NC, NS = 2, 16
NW = NC * NS


def _sc_embed(node_idx, edge_idx, node_table, edge_table, node_pos, edge_pos):
    mesh = plsc.VectorSubcoreMesh(core_axis_name="c", subcore_axis_name="s")

    @functools.partial(
        pl.kernel,
        mesh=mesh,
        compiler_params=pltpu.CompilerParams(use_tc_tiling_on_sc=False),
        out_type=jax.ShapeDtypeStruct((2 * WALK_LEN * BATCH, D), jnp.float32),
        scratch_types=[
            pltpu.VMEM((C,), jnp.int32),
            pltpu.VMEM((C, D), jnp.float32),
            pltpu.VMEM((WALK_LEN, D), jnp.float32),
            pltpu.VMEM((WALK_LEN, D), jnp.float32),
            pltpu.SemaphoreType.DMA,
        ],
    )
    def k(nidx_hbm, eidx_hbm, ntab_hbm, etab_hbm, npos_hbm, epos_hbm,
          out_hbm, idx_v, rows_v, npos_v, epos_v, sem):
        wid = lax.axis_index("s") * NC + lax.axis_index("c")
        pltpu.sync_copy(npos_hbm, npos_v)
        pltpu.sync_copy(epos_hbm, epos_v)

        def do_table(idx_hbm, tab_hbm, pos_v, out_row_off):
            def body(pos, _):
                base = pos * BATCH + wid * C
                pltpu.sync_copy(idx_hbm.at[pl.ds(base, C)], idx_v)
                pltpu.async_copy(tab_hbm.at[idx_v], rows_v, sem).wait()
                pos_vecs = [pos_v[pos, pl.ds(16 * d4, 16)]
                            for d4 in range(D // 16)]

                def row_body(r, _):
                    for d4 in range(D // 16):
                        sl = pl.ds(16 * d4, 16)
                        rows_v[r, sl] = rows_v[r, sl] + pos_vecs[d4]
                    return 0

                lax.fori_loop(0, C, row_body, 0)
                pltpu.sync_copy(rows_v, out_hbm.at[pl.ds(out_row_off + base, C)])
                return 0

            lax.fori_loop(0, WALK_LEN, body, 0)

        do_table(nidx_hbm, ntab_hbm, npos_v, 0)
        do_table(eidx_hbm, etab_hbm, epos_v, WALK_LEN * BATCH)

    return k(node_idx, edge_idx, node_table, edge_table, node_pos, edge_pos)


def kernel(node_idx, edge_idx, node_table, edge_table, node_pos, edge_pos):
    nflat = node_idx.reshape(-1).astype(jnp.int32)
    eflat = edge_idx.reshape(-1).astype(jnp.int32)
    out = _sc_embed(nflat, eflat, node_table, edge_table, node_pos, edge_pos)
    return out.reshape(2 * WALK_LEN, BATCH, D)
